# baseline (speedup 1.0000x reference)
import jax
import jax.numpy as jnp
from jax.experimental import pallas as pl
from jax.experimental.pallas import tpu as pltpu

_ROWS = 1056


def _fused_body(h_ref, w_ref, mem_ref, o_ref):
    q = jnp.dot(h_ref[...], w_ref[...], preferred_element_type=jnp.float32)
    s = jax.lax.dot_general(
        q, mem_ref[...], (((1,), (1,)), ((), ())),
        preferred_element_type=jnp.float32)
    m = jnp.max(s, axis=-1, keepdims=True)
    e = jnp.exp(s - m)
    p = e / jnp.sum(e, axis=-1, keepdims=True)
    o_ref[...] = jnp.dot(p, mem_ref[...], preferred_element_type=jnp.float32)


def kernel(x, x_mark, Memory, Wq0, Wq1, Wq2, Wq3, Wq4, We1, We2):
    B, C, L = x.shape
    Cm = x_mark.shape[-1]
    mem_num, mem_dim = Memory.shape
    h = jnp.concatenate([x, jnp.transpose(x_mark, (0, 2, 1))], axis=1)
    rows = B * (C + Cm)
    h2 = h.reshape(rows, L)
    grid = rows // _ROWS
    out = pl.pallas_call(
        _fused_body,
        grid=(grid,),
        in_specs=[
            pl.BlockSpec((_ROWS, L), lambda i: (i, 0)),
            pl.BlockSpec((L, mem_dim), lambda i: (0, 0)),
            pl.BlockSpec((mem_num, mem_dim), lambda i: (0, 0)),
        ],
        out_specs=pl.BlockSpec((_ROWS, mem_dim), lambda i: (i, 0)),
        out_shape=jax.ShapeDtypeStruct((rows, mem_dim), jnp.float32),
        compiler_params=pltpu.CompilerParams(
            dimension_semantics=("arbitrary",)),
    )(h2, Wq4, Memory)
    return out.reshape(B, C + Cm, mem_dim)


# speedup vs baseline: 2.4383x; 2.4383x over previous
import jax
import jax.numpy as jnp
from jax.experimental import pallas as pl
from jax.experimental.pallas import tpu as pltpu

_BB = 8


def _fused_body(x_ref, xm_ref, w_ref, mem_ref, o_ref):
    bb, c, l = x_ref.shape
    cm = xm_ref.shape[1]
    mem = mem_ref[...]
    w = w_ref[...]

    def attend(h):
        q = jnp.dot(h, w, preferred_element_type=jnp.float32)
        s = jax.lax.dot_general(
            q, mem, (((1,), (1,)), ((), ())),
            preferred_element_type=jnp.float32)
        m = jnp.max(s, axis=-1, keepdims=True)
        e = jnp.exp(s - m)
        p = e / jnp.sum(e, axis=-1, keepdims=True)
        return jnp.dot(p, mem, preferred_element_type=jnp.float32)

    vx = attend(x_ref[...].reshape(bb * c, l))
    vm = attend(xm_ref[...].reshape(bb * cm, l))
    o_ref[:, :c, :] = vx.reshape(bb, c, -1)
    o_ref[:, c:, :] = vm.reshape(bb, cm, -1)


def kernel(x, x_mark, Memory, Wq0, Wq1, Wq2, Wq3, Wq4, We1, We2):
    B, C, L = x.shape
    Cm = x_mark.shape[-1]
    mem_num, mem_dim = Memory.shape
    xmt = jnp.transpose(x_mark, (0, 2, 1))
    grid = B // _BB
    out = pl.pallas_call(
        _fused_body,
        grid=(grid,),
        in_specs=[
            pl.BlockSpec((_BB, C, L), lambda i: (i, 0, 0)),
            pl.BlockSpec((_BB, Cm, L), lambda i: (i, 0, 0)),
            pl.BlockSpec((L, mem_dim), lambda i: (0, 0)),
            pl.BlockSpec((mem_num, mem_dim), lambda i: (0, 0)),
        ],
        out_specs=pl.BlockSpec((_BB, C + Cm, mem_dim), lambda i: (i, 0, 0)),
        out_shape=jax.ShapeDtypeStruct((B, C + Cm, mem_dim), jnp.float32),
        compiler_params=pltpu.CompilerParams(
            dimension_semantics=("arbitrary",)),
    )(x, xmt, Wq4, Memory)
    return out
